# Initial kernel scaffold; baseline (speedup 1.0000x reference)
import jax
import jax.numpy as jnp
from jax.experimental import pallas as pl


def kernel(cons_x, edge_index, edge_attr, var_x, params):
    raise NotImplementedError("write your pallas kernel here")


# speedup vs baseline: 24.6424x; 24.6424x over previous
import functools

import jax
import jax.numpy as jnp
from jax import lax
from jax.experimental import pallas as pl
from jax.experimental.pallas import tpu as pltpu
from jax.experimental.pallas import tpu_sc as plsc

_H = 16
_NC = 2
_NS = 16
_NW = _NC * _NS
_C = 512


def _relu(x):
    return jnp.maximum(x, 0.0)


def _linear(x, w, b=None):
    y = x @ w
    if b is not None:
        y = y + b
    return y


def _layer_norm(x, g, b, eps=1e-5):
    m = jnp.mean(x, axis=-1, keepdims=True)
    v = jnp.var(x, axis=-1, keepdims=True)
    return (x - m) / jnp.sqrt(v + eps) * g + b


@functools.lru_cache(maxsize=None)
def _make_edge_pass(n_pad, e_pad):
    epw = e_pad // _NW
    nch = epw // _C
    trows = n_pad // _NS
    nz = trows // _C
    zrem = trows % _C
    mesh = plsc.VectorSubcoreMesh(core_axis_name="c", subcore_axis_name="s")

    @functools.partial(
        pl.kernel,
        out_type=(
            jax.ShapeDtypeStruct((_NC, n_pad, _H), jnp.float32),
            jax.ShapeDtypeStruct((_NC, n_pad), jnp.float32),
        ),
        mesh=mesh,
        scratch_types=[
            pltpu.VMEM_SHARED((n_pad, _H), jnp.float32),
            pltpu.VMEM_SHARED((n_pad,), jnp.float32),
            pltpu.VMEM((_C,), jnp.int32),
            pltpu.VMEM((_C,), jnp.int32),
            pltpu.VMEM((_C,), jnp.float32),
            pltpu.VMEM((_C, _H), jnp.float32),
            pltpu.VMEM((_C, _H), jnp.float32),
            pltpu.VMEM((_C,), jnp.float32),
            pltpu.VMEM((_H,), jnp.float32),
            pltpu.VMEM((_H,), jnp.float32),
            pltpu.VMEM((_H,), jnp.float32),
            pltpu.SemaphoreType.DMA,
            pltpu.SemaphoreType.DMA,
        ],
        compiler_params=pltpu.CompilerParams(use_tc_tiling_on_sc=False),
    )
    def edge_pass(a_hbm, b_hbm, li_hbm, ri_hbm, ea_hbm, fe_hbm, w_hbm,
                  bb_hbm, acc_out, deg_out,
                  acc_sh, deg_sh, li_v, ri_v, ea_v, a_v, b_v, ones_v,
                  fe_v, w_v, bb_v, sem_a, sem_b):
        cid = lax.axis_index("c")
        sid = lax.axis_index("s")
        wid = sid * _NC + cid

        pltpu.sync_copy(fe_hbm, fe_v)
        pltpu.sync_copy(w_hbm, w_v)
        pltpu.sync_copy(bb_hbm, bb_v)

        zeros16 = jnp.zeros((_H,), jnp.float32)
        ones16 = jnp.ones((_H,), jnp.float32)

        def fill_z(i, _):
            a_v[i, :] = zeros16
            ea_v[pl.ds((i % (_C // _H)) * _H, _H)] = zeros16
            ones_v[pl.ds((i % (_C // _H)) * _H, _H)] = ones16
            return 0

        lax.fori_loop(0, _C, fill_z, 0)

        base = sid * trows
        for j in range(nz):
            pltpu.sync_copy(a_v, acc_sh.at[pl.ds(base + j * _C, _C)])
            pltpu.sync_copy(ea_v, deg_sh.at[pl.ds(base + j * _C, _C)])
        if zrem:
            pltpu.sync_copy(a_v.at[pl.ds(0, zrem)],
                            acc_sh.at[pl.ds(base + nz * _C, zrem)])
            pltpu.sync_copy(ea_v.at[pl.ds(0, zrem)],
                            deg_sh.at[pl.ds(base + nz * _C, zrem)])
        plsc.subcore_barrier()

        fe = fe_v[...]
        wv = w_v[...]
        bv = bb_v[...]
        ebase = wid * epw

        def chunk(kc, _):
            off = ebase + kc * _C
            pltpu.sync_copy(li_hbm.at[pl.ds(off, _C)], li_v)
            pltpu.sync_copy(ri_hbm.at[pl.ds(off, _C)], ri_v)
            pltpu.sync_copy(ea_hbm.at[pl.ds(off, _C)], ea_v)
            ga = pltpu.async_copy(a_hbm.at[li_v], a_v, sem_a)
            gb = pltpu.async_copy(b_hbm.at[ri_v], b_v, sem_b)
            ga.wait()
            gb.wait()

            def group(g, _):
                s16 = _relu(ea_v[pl.ds(g * _H, _H)] * wv + bv)
                for j in range(_H):
                    i = g * _H + j
                    sj = s16.at[jnp.full((_H,), j, jnp.int32)].get(
                        mode="promise_in_bounds")
                    a_v[i, :] = _relu(a_v[i, :] + b_v[i, :] + sj * fe)
                return 0

            lax.fori_loop(0, _C // _H, group, 0)
            pltpu.sync_copy(a_v, acc_sh.at[ri_v], add=True)
            pltpu.sync_copy(ones_v, deg_sh.at[ri_v], add=True)
            return 0

        lax.fori_loop(0, nch, chunk, 0)
        plsc.subcore_barrier()

        pltpu.sync_copy(acc_sh.at[pl.ds(base, trows)],
                        acc_out.at[cid].at[pl.ds(base, trows)])
        pltpu.sync_copy(deg_sh.at[pl.ds(base, trows)],
                        deg_out.at[cid].at[pl.ds(base, trows)])

    return edge_pass


def _edge_stage(a_tab, b_tab, li, ri, ea, fe, w, b, n):
    e = li.shape[0]
    n_pad = ((n + 1 + _NS * 128 - 1) // (_NS * 128)) * (_NS * 128)
    e_pad = ((e + _NW * _C - 1) // (_NW * _C)) * (_NW * _C)
    if e_pad != e:
        pad = e_pad - e
        dummy = n + (jnp.arange(pad, dtype=jnp.int32) % (n_pad - n))
        li = jnp.concatenate([li, dummy])
        ri = jnp.concatenate([ri, dummy])
        ea = jnp.concatenate([ea, jnp.zeros((pad,), jnp.float32)])
        a_tab = jnp.pad(a_tab, ((0, n_pad - n), (0, 0)))
        b_tab = jnp.pad(b_tab, ((0, n_pad - n), (0, 0)))
    fe16 = jnp.broadcast_to(fe.reshape(-1), (_H,)).astype(jnp.float32)
    w16 = jnp.full((_H,), w, jnp.float32)
    b16 = jnp.full((_H,), b, jnp.float32)
    kern = _make_edge_pass(n_pad, e_pad)
    acc, deg = kern(a_tab, b_tab, li, ri, ea, fe16, w16, b16)
    s = acc[0, :n, :] + acc[1, :n, :]
    d = deg[0, :n] + deg[1, :n]
    return s, d


def _bgc_node_post(s, deg, right, p):
    agg = s @ p['ff_W'] + deg[:, None] * p['ff_b']
    post = _relu(agg) @ p['pc_W'] + p['pc_b']
    h2 = _relu(post @ p['o1_W'][:_H] + right @ p['o1_W'][_H:] + p['o1_b'])
    return h2 @ p['o2_W'] + p['o2_b']


def _trans_conv(x, p):
    n = x.shape[0]
    x = _linear(x, p['fc_W'], p['fc_b'])
    x = _layer_norm(x, p['ln0_g'], p['ln0_b'])
    x = _relu(x)
    prev = x
    q = _linear(x, p['Wq'], p['bq'])
    k = _linear(x, p['Wk'], p['bk'])
    v = _linear(x, p['Wv'], p['bv'])
    q = q / (jnp.linalg.norm(q) + 1e-8)
    k = k / (jnp.linalg.norm(k) + 1e-8)
    kv = k.T @ v
    num = q @ kv + jnp.float32(n) * v
    denom = q @ jnp.sum(k, axis=0)[:, None] + jnp.float32(n)
    attn = num / denom
    x = (attn + prev) / 2.0
    x = _layer_norm(x, p['ln1_g'], p['ln1_b'])
    return _relu(x)


def kernel(cons_x, edge_index, edge_attr, var_x, params):
    p = params
    g = p['gnn']
    n_cons = cons_x.shape[0]
    n_var = var_x.shape[0]

    var_emb = _linear(var_x, p['ve_W'], p['ve_b'])
    cons_emb = _linear(cons_x, p['ce_W'], p['ce_b'])

    x1 = _trans_conv(var_emb, p['trans'])

    cons_h = _relu(_linear(_relu(_linear(cons_emb, g['ce1_W'], g['ce1_b'])),
                           g['ce2_W'], g['ce2_b']))
    var_h = _relu(_linear(_relu(_linear(var_emb, g['ve1_W'], g['ve1_b'])),
                          g['ve2_W'], g['ve2_b']))

    ei0 = edge_index[0].astype(jnp.int32)
    ei1 = edge_index[1].astype(jnp.int32)
    ea = edge_attr[:, 0].astype(jnp.float32)

    v2c = g['v2c']
    a1 = var_h @ v2c['fl_W'] + v2c['fl_b']
    b1 = cons_h @ v2c['fr_W']
    s1, d1 = _edge_stage(a1, b1, ei1, ei0, ea, v2c['fe_W'],
                         g['ee_W'][0, 0], g['ee_b'][0], n_cons)
    cons2 = _bgc_node_post(s1, d1, cons_h, v2c)

    c2v = g['c2v']
    a2 = cons2 @ c2v['fl_W'] + c2v['fl_b']
    b2 = var_h @ c2v['fr_W']
    s2, d2 = _edge_stage(a2, b2, ei0, ei1, ea, c2v['fe_W'],
                         g['ee_W'][0, 0], g['ee_b'][0], n_var)
    var2 = _bgc_node_post(s2, d2, var_h, c2v)

    var_g = _relu(_linear(var2, g['ov_W'], g['ov_b']))
    cons_g = _relu(_linear(cons2, g['oc_W'], g['oc_b']))

    x = jnp.concatenate([0.8 * var_g, 0.2 * x1], axis=-1)
    out = _linear(x, p['fc_W'], p['fc_b'])
    return out, var_g, cons_g


# speedup vs baseline: 33.9219x; 1.3766x over previous
import functools

import jax
import jax.numpy as jnp
from jax import lax
from jax.experimental import pallas as pl
from jax.experimental.pallas import tpu as pltpu
from jax.experimental.pallas import tpu_sc as plsc

_H = 16
_NC = 2
_NS = 16
_NW = _NC * _NS
_C = 256
_NBUF = 3


def _relu(x):
    return jnp.maximum(x, 0.0)


def _linear(x, w, b=None):
    y = x @ w
    if b is not None:
        y = y + b
    return y


def _layer_norm(x, g, b, eps=1e-5):
    m = jnp.mean(x, axis=-1, keepdims=True)
    v = jnp.var(x, axis=-1, keepdims=True)
    return (x - m) / jnp.sqrt(v + eps) * g + b


@functools.lru_cache(maxsize=None)
def _make_edge_pass(n_pad, e_pad):
    epw = e_pad // _NW
    nch = epw // _C
    trows = n_pad // _NS
    nz = trows // _C
    zrem = trows % _C
    mesh = plsc.VectorSubcoreMesh(core_axis_name="c", subcore_axis_name="s")

    vec_t = pltpu.VMEM((_H,), jnp.float32)
    idx_t = pltpu.VMEM((_C,), jnp.int32)
    sca_t = pltpu.VMEM((_C,), jnp.float32)
    row_t = pltpu.VMEM((_C, _H), jnp.float32)

    @functools.partial(
        pl.kernel,
        out_type=jax.ShapeDtypeStruct((_NC, n_pad, _H), jnp.float32),
        mesh=mesh,
        scratch_types=(
            [pltpu.VMEM_SHARED((n_pad, _H), jnp.float32)]
            + [idx_t] * _NBUF + [idx_t] * _NBUF + [sca_t] * _NBUF
            + [row_t] * _NBUF + [row_t] * _NBUF
            + [vec_t] * 4
            + [pltpu.SemaphoreType.DMA] * (3 * _NBUF)
        ),
        compiler_params=pltpu.CompilerParams(use_tc_tiling_on_sc=False),
    )
    def edge_pass(a_hbm, b_hbm, li_hbm, ri_hbm, ea_hbm, fe_hbm, w_hbm,
                  bb_hbm, t_hbm, acc_out,
                  acc_sh,
                  li0, li1, li2, ri0, ri1, ri2, ea0, ea1, ea2,
                  av0, av1, av2, bv0, bv1, bv2,
                  fe_v, w_v, bb_v, t_v,
                  si0, si1, si2, sg0, sg1, sg2, ss0, ss1, ss2):
        li_v = (li0, li1, li2)
        ri_v = (ri0, ri1, ri2)
        ea_v = (ea0, ea1, ea2)
        a_v = (av0, av1, av2)
        b_v = (bv0, bv1, bv2)
        si = (si0, si1, si2)
        sg = (sg0, sg1, sg2)
        ss = (ss0, ss1, ss2)

        cid = lax.axis_index("c")
        sid = lax.axis_index("s")
        wid = sid * _NC + cid
        ebase = wid * epw

        pltpu.sync_copy(fe_hbm, fe_v)
        pltpu.sync_copy(w_hbm, w_v)
        pltpu.sync_copy(bb_hbm, bb_v)
        pltpu.sync_copy(t_hbm, t_v)

        zeros16 = jnp.zeros((_H,), jnp.float32)

        def fill_z(i, _):
            av0[i, :] = zeros16
            return 0

        lax.fori_loop(0, _C, fill_z, 0)

        base = sid * trows
        for j in range(nz):
            pltpu.sync_copy(av0, acc_sh.at[pl.ds(base + j * _C, _C)])
        if zrem:
            pltpu.sync_copy(av0.at[pl.ds(0, zrem)],
                            acc_sh.at[pl.ds(base + nz * _C, zrem)])
        plsc.subcore_barrier()

        fe = fe_v[...]
        wv = w_v[...]
        bv = bb_v[...]
        tv = t_v[...]

        def idx_issue(k, b):
            off = ebase + k * _C
            pltpu.async_copy(li_hbm.at[pl.ds(off, _C)], li_v[b], si[b])
            pltpu.async_copy(ri_hbm.at[pl.ds(off, _C)], ri_v[b], si[b])
            pltpu.async_copy(ea_hbm.at[pl.ds(off, _C)], ea_v[b], si[b])

        def idx_wait(k, b):
            off = ebase + k * _C
            pltpu.make_async_copy(li_hbm.at[pl.ds(off, _C)], li_v[b],
                                  si[b]).wait()
            pltpu.make_async_copy(ri_hbm.at[pl.ds(off, _C)], ri_v[b],
                                  si[b]).wait()
            pltpu.make_async_copy(ea_hbm.at[pl.ds(off, _C)], ea_v[b],
                                  si[b]).wait()

        def gath_issue(b):
            pltpu.async_copy(a_hbm.at[li_v[b]], a_v[b], sg[b])
            pltpu.async_copy(b_hbm.at[ri_v[b]], b_v[b], sg[b])

        def gath_wait(b):
            pltpu.make_async_copy(a_hbm.at[li_v[b]], a_v[b], sg[b]).wait()
            pltpu.make_async_copy(b_hbm.at[ri_v[b]], b_v[b], sg[b]).wait()

        def scat_issue(b):
            pltpu.async_copy(a_v[b], acc_sh.at[ri_v[b]], ss[b], add=True)

        def scat_wait(b):
            pltpu.make_async_copy(a_v[b], acc_sh.at[ri_v[b]], ss[b]).wait()

        def compute(b):
            av = a_v[b]
            bvr = b_v[b]
            eav = ea_v[b]

            def group(g, _):
                s16 = _relu(eav[pl.ds(g * _H, _H)] * wv + bv)
                for j in range(_H):
                    i = g * _H + j
                    sj = s16.at[jnp.full((_H,), j, jnp.int32)].get(
                        mode="promise_in_bounds")
                    av[i, :] = _relu(av[i, :] + bvr[i, :] + sj * fe) + tv
                return 0

            lax.fori_loop(0, _C // _H, group, 0)

        idx_issue(0, 0)
        idx_issue(1, 1)
        idx_wait(0, 0)
        gath_issue(0)

        def outer(g, _):
            for j in range(_NBUF):
                k = g * _NBUF + j
                b = j
                b1 = (j + 1) % _NBUF
                b2 = (j + 2) % _NBUF
                gath_wait(b)

                @pl.when(k >= 1)
                def _():
                    scat_wait(b2)

                @pl.when(k + 2 < nch)
                def _():
                    idx_issue(k + 2, b2)

                @pl.when(k + 1 < nch)
                def _():
                    idx_wait(k + 1, b1)
                    gath_issue(b1)

                compute(b)
                scat_issue(b)
            return 0

        lax.fori_loop(0, nch // _NBUF, outer, 0)
        scat_wait((nch - 1) % _NBUF)
        plsc.subcore_barrier()

        pltpu.sync_copy(acc_sh.at[pl.ds(base, trows)],
                        acc_out.at[cid].at[pl.ds(base, trows)])

    return edge_pass


def _edge_stage(a_tab, b_tab, li, ri, ea, fe, w, b, t, n):
    e = li.shape[0]
    n_pad = ((n + 1 + _NS * 128 - 1) // (_NS * 128)) * (_NS * 128)
    quant = _NW * _C * _NBUF
    e_pad = ((e + quant - 1) // quant) * quant
    if e_pad != e:
        pad = e_pad - e
        dummy = n + (jnp.arange(pad, dtype=jnp.int32) % (n_pad - n))
        li = jnp.concatenate([li, dummy])
        ri = jnp.concatenate([ri, dummy])
        ea = jnp.concatenate([ea, jnp.zeros((pad,), jnp.float32)])
        a_tab = jnp.pad(a_tab, ((0, n_pad - n), (0, 0)))
        b_tab = jnp.pad(b_tab, ((0, n_pad - n), (0, 0)))
    fe16 = jnp.broadcast_to(fe.reshape(-1), (_H,)).astype(jnp.float32)
    w16 = jnp.full((_H,), w, jnp.float32)
    b16 = jnp.full((_H,), b, jnp.float32)
    kern = _make_edge_pass(n_pad, e_pad)
    acc = kern(a_tab, b_tab, li, ri, ea, fe16, w16, b16, t)
    return acc[0, :n, :] + acc[1, :n, :]


def _bgc_node_post(s, right, p):
    agg = s @ p['ff_W']
    post = _relu(agg) @ p['pc_W'] + p['pc_b']
    h2 = _relu(post @ p['o1_W'][:_H] + right @ p['o1_W'][_H:] + p['o1_b'])
    return h2 @ p['o2_W'] + p['o2_b']


def _bias_fold(p):
    ff_b = p['ff_b']
    t = jnp.linalg.solve(p['ff_W'].T, ff_b)
    return jnp.where(jnp.any(jnp.abs(ff_b) > 0), t, jnp.zeros((_H,)))


def _trans_conv(x, p):
    n = x.shape[0]
    x = _linear(x, p['fc_W'], p['fc_b'])
    x = _layer_norm(x, p['ln0_g'], p['ln0_b'])
    x = _relu(x)
    prev = x
    q = _linear(x, p['Wq'], p['bq'])
    k = _linear(x, p['Wk'], p['bk'])
    v = _linear(x, p['Wv'], p['bv'])
    q = q / (jnp.linalg.norm(q) + 1e-8)
    k = k / (jnp.linalg.norm(k) + 1e-8)
    kv = k.T @ v
    num = q @ kv + jnp.float32(n) * v
    denom = q @ jnp.sum(k, axis=0)[:, None] + jnp.float32(n)
    attn = num / denom
    x = (attn + prev) / 2.0
    x = _layer_norm(x, p['ln1_g'], p['ln1_b'])
    return _relu(x)


def kernel(cons_x, edge_index, edge_attr, var_x, params):
    p = params
    g = p['gnn']
    n_cons = cons_x.shape[0]
    n_var = var_x.shape[0]

    var_emb = _linear(var_x, p['ve_W'], p['ve_b'])
    cons_emb = _linear(cons_x, p['ce_W'], p['ce_b'])

    x1 = _trans_conv(var_emb, p['trans'])

    cons_h = _relu(_linear(_relu(_linear(cons_emb, g['ce1_W'], g['ce1_b'])),
                           g['ce2_W'], g['ce2_b']))
    var_h = _relu(_linear(_relu(_linear(var_emb, g['ve1_W'], g['ve1_b'])),
                          g['ve2_W'], g['ve2_b']))

    ei0 = edge_index[0].astype(jnp.int32)
    ei1 = edge_index[1].astype(jnp.int32)
    ea = edge_attr[:, 0].astype(jnp.float32)

    v2c = g['v2c']
    a1 = var_h @ v2c['fl_W'] + v2c['fl_b']
    b1 = cons_h @ v2c['fr_W']
    s1 = _edge_stage(a1, b1, ei1, ei0, ea, v2c['fe_W'],
                     g['ee_W'][0, 0], g['ee_b'][0], _bias_fold(v2c), n_cons)
    cons2 = _bgc_node_post(s1, cons_h, v2c)

    c2v = g['c2v']
    a2 = cons2 @ c2v['fl_W'] + c2v['fl_b']
    b2 = var_h @ c2v['fr_W']
    s2 = _edge_stage(a2, b2, ei0, ei1, ea, c2v['fe_W'],
                     g['ee_W'][0, 0], g['ee_b'][0], _bias_fold(c2v), n_var)
    var2 = _bgc_node_post(s2, var_h, c2v)

    var_g = _relu(_linear(var2, g['ov_W'], g['ov_b']))
    cons_g = _relu(_linear(cons2, g['oc_W'], g['oc_b']))

    x = jnp.concatenate([0.8 * var_g, 0.2 * x1], axis=-1)
    out = _linear(x, p['fc_W'], p['fc_b'])
    return out, var_g, cons_g


# speedup vs baseline: 34.3294x; 1.0120x over previous
import functools

import jax
import jax.numpy as jnp
from jax import lax
from jax.experimental import pallas as pl
from jax.experimental.pallas import tpu as pltpu
from jax.experimental.pallas import tpu_sc as plsc

_H = 16
_NC = 2
_NS = 16
_NW = _NC * _NS
_C = 256
_NBUF = 3


def _relu(x):
    return jnp.maximum(x, 0.0)


def _linear(x, w, b=None):
    y = x @ w
    if b is not None:
        y = y + b
    return y


def _layer_norm(x, g, b, eps=1e-5):
    m = jnp.mean(x, axis=-1, keepdims=True)
    v = jnp.var(x, axis=-1, keepdims=True)
    return (x - m) / jnp.sqrt(v + eps) * g + b


@functools.lru_cache(maxsize=None)
def _make_edge_pass(n_pad, e_pad):
    epw = e_pad // _NW
    nch = epw // _C
    trows = n_pad // _NS
    nz = trows // _C
    zrem = trows % _C
    mesh = plsc.VectorSubcoreMesh(core_axis_name="c", subcore_axis_name="s")

    vec_t = pltpu.VMEM((_H,), jnp.float32)
    idx_t = pltpu.VMEM((_C,), jnp.int32)
    sca_t = pltpu.VMEM((_C,), jnp.float32)
    row_t = pltpu.VMEM((_C, _H), jnp.float32)

    @functools.partial(
        pl.kernel,
        out_type=jax.ShapeDtypeStruct((_NC, n_pad, _H), jnp.float32),
        mesh=mesh,
        scratch_types=(
            [pltpu.VMEM_SHARED((n_pad, _H), jnp.float32)]
            + [idx_t] * _NBUF + [idx_t] * _NBUF + [sca_t] * _NBUF
            + [row_t] * _NBUF + [row_t] * _NBUF
            + [vec_t] * 4
            + [pltpu.SemaphoreType.DMA] * (3 * _NBUF)
        ),
        compiler_params=pltpu.CompilerParams(use_tc_tiling_on_sc=False),
    )
    def edge_pass(a_hbm, b_hbm, li_hbm, ri_hbm, ea_hbm, fe_hbm, w_hbm,
                  bb_hbm, t_hbm, acc_out,
                  acc_sh,
                  li0, li1, li2, ri0, ri1, ri2, ea0, ea1, ea2,
                  av0, av1, av2, bv0, bv1, bv2,
                  fe_v, w_v, bb_v, t_v,
                  si0, si1, si2, sg0, sg1, sg2, ss0, ss1, ss2):
        li_v = (li0, li1, li2)
        ri_v = (ri0, ri1, ri2)
        ea_v = (ea0, ea1, ea2)
        a_v = (av0, av1, av2)
        b_v = (bv0, bv1, bv2)
        si = (si0, si1, si2)
        sg = (sg0, sg1, sg2)
        ss = (ss0, ss1, ss2)

        cid = lax.axis_index("c")
        sid = lax.axis_index("s")
        wid = sid * _NC + cid
        ebase = wid * epw

        pltpu.sync_copy(fe_hbm, fe_v)
        pltpu.sync_copy(w_hbm, w_v)
        pltpu.sync_copy(bb_hbm, bb_v)
        pltpu.sync_copy(t_hbm, t_v)

        zeros16 = jnp.zeros((_H,), jnp.float32)

        def fill_z(i, _):
            av0[i, :] = zeros16
            return 0

        lax.fori_loop(0, _C, fill_z, 0)

        base = sid * trows
        for j in range(nz):
            pltpu.sync_copy(av0, acc_sh.at[pl.ds(base + j * _C, _C)])
        if zrem:
            pltpu.sync_copy(av0.at[pl.ds(0, zrem)],
                            acc_sh.at[pl.ds(base + nz * _C, zrem)])
        plsc.subcore_barrier()

        fe = fe_v[...]
        wv = w_v[...]
        bv = bb_v[...]
        tv = t_v[...]

        def idx_issue(k, b):
            off = ebase + k * _C
            pltpu.async_copy(li_hbm.at[pl.ds(off, _C)], li_v[b], si[b])
            pltpu.async_copy(ri_hbm.at[pl.ds(off, _C)], ri_v[b], si[b])
            pltpu.async_copy(ea_hbm.at[pl.ds(off, _C)], ea_v[b], si[b])

        def idx_wait(k, b):
            off = ebase + k * _C
            pltpu.make_async_copy(li_hbm.at[pl.ds(off, _C)], li_v[b],
                                  si[b]).wait()
            pltpu.make_async_copy(ri_hbm.at[pl.ds(off, _C)], ri_v[b],
                                  si[b]).wait()
            pltpu.make_async_copy(ea_hbm.at[pl.ds(off, _C)], ea_v[b],
                                  si[b]).wait()

        def gath_issue(b):
            pltpu.async_copy(a_hbm.at[li_v[b]], a_v[b], sg[b])
            pltpu.async_copy(b_hbm.at[ri_v[b]], b_v[b], sg[b])

        def gath_wait(b):
            pltpu.make_async_copy(a_hbm.at[li_v[b]], a_v[b], sg[b]).wait()
            pltpu.make_async_copy(b_hbm.at[ri_v[b]], b_v[b], sg[b]).wait()

        def scat_issue(b):
            pltpu.async_copy(a_v[b], acc_sh.at[ri_v[b]], ss[b], add=True)

        def scat_wait(b):
            pltpu.make_async_copy(a_v[b], acc_sh.at[ri_v[b]], ss[b]).wait()

        def compute(b):
            av = a_v[b]
            bvr = b_v[b]
            eav = ea_v[b]

            @functools.partial(plsc.parallel_loop, 0, _C // _H, unroll=2)
            def group(g):
                s16 = _relu(eav[pl.ds(g * _H, _H)] * wv + bv)
                for j in range(_H):
                    i = g * _H + j
                    sj = s16.at[jnp.full((_H,), j, jnp.int32)].get(
                        mode="promise_in_bounds")
                    av[i, :] = _relu(av[i, :] + bvr[i, :] + sj * fe) + tv

        idx_issue(0, 0)
        idx_issue(1, 1)
        idx_wait(0, 0)
        gath_issue(0)

        def outer(g, _):
            for j in range(_NBUF):
                k = g * _NBUF + j
                b = j
                b1 = (j + 1) % _NBUF
                b2 = (j + 2) % _NBUF
                gath_wait(b)

                @pl.when(k >= 1)
                def _():
                    scat_wait(b2)

                @pl.when(k + 2 < nch)
                def _():
                    idx_issue(k + 2, b2)

                @pl.when(k + 1 < nch)
                def _():
                    idx_wait(k + 1, b1)
                    gath_issue(b1)

                compute(b)
                scat_issue(b)
            return 0

        lax.fori_loop(0, nch // _NBUF, outer, 0)
        scat_wait((nch - 1) % _NBUF)
        plsc.subcore_barrier()

        pltpu.sync_copy(acc_sh.at[pl.ds(base, trows)],
                        acc_out.at[cid].at[pl.ds(base, trows)])

    return edge_pass


def _edge_stage(a_tab, b_tab, li, ri, ea, fe, w, b, t, n):
    e = li.shape[0]
    n_pad = ((n + 1 + _NS * 128 - 1) // (_NS * 128)) * (_NS * 128)
    quant = _NW * _C * _NBUF
    e_pad = ((e + quant - 1) // quant) * quant
    if e_pad != e:
        pad = e_pad - e
        dummy = n + (jnp.arange(pad, dtype=jnp.int32) % (n_pad - n))
        li = jnp.concatenate([li, dummy])
        ri = jnp.concatenate([ri, dummy])
        ea = jnp.concatenate([ea, jnp.zeros((pad,), jnp.float32)])
        a_tab = jnp.pad(a_tab, ((0, n_pad - n), (0, 0)))
        b_tab = jnp.pad(b_tab, ((0, n_pad - n), (0, 0)))
    fe16 = jnp.broadcast_to(fe.reshape(-1), (_H,)).astype(jnp.float32)
    w16 = jnp.full((_H,), w, jnp.float32)
    b16 = jnp.full((_H,), b, jnp.float32)
    kern = _make_edge_pass(n_pad, e_pad)
    acc = kern(a_tab, b_tab, li, ri, ea, fe16, w16, b16, t)
    return acc[0, :n, :] + acc[1, :n, :]


def _bgc_node_post(s, right, p):
    agg = s @ p['ff_W']
    post = _relu(agg) @ p['pc_W'] + p['pc_b']
    h2 = _relu(post @ p['o1_W'][:_H] + right @ p['o1_W'][_H:] + p['o1_b'])
    return h2 @ p['o2_W'] + p['o2_b']


def _bias_fold(p):
    ff_b = p['ff_b']
    return lax.cond(jnp.any(jnp.abs(ff_b) > 0),
                    lambda: jnp.linalg.solve(p['ff_W'].T, ff_b),
                    lambda: jnp.zeros((_H,), jnp.float32))


def _trans_conv(x, p):
    n = x.shape[0]
    x = _linear(x, p['fc_W'], p['fc_b'])
    x = _layer_norm(x, p['ln0_g'], p['ln0_b'])
    x = _relu(x)
    prev = x
    q = _linear(x, p['Wq'], p['bq'])
    k = _linear(x, p['Wk'], p['bk'])
    v = _linear(x, p['Wv'], p['bv'])
    q = q / (jnp.linalg.norm(q) + 1e-8)
    k = k / (jnp.linalg.norm(k) + 1e-8)
    kv = k.T @ v
    num = q @ kv + jnp.float32(n) * v
    denom = q @ jnp.sum(k, axis=0)[:, None] + jnp.float32(n)
    attn = num / denom
    x = (attn + prev) / 2.0
    x = _layer_norm(x, p['ln1_g'], p['ln1_b'])
    return _relu(x)


def kernel(cons_x, edge_index, edge_attr, var_x, params):
    p = params
    g = p['gnn']
    n_cons = cons_x.shape[0]
    n_var = var_x.shape[0]

    var_emb = _linear(var_x, p['ve_W'], p['ve_b'])
    cons_emb = _linear(cons_x, p['ce_W'], p['ce_b'])

    x1 = _trans_conv(var_emb, p['trans'])

    cons_h = _relu(_linear(_relu(_linear(cons_emb, g['ce1_W'], g['ce1_b'])),
                           g['ce2_W'], g['ce2_b']))
    var_h = _relu(_linear(_relu(_linear(var_emb, g['ve1_W'], g['ve1_b'])),
                          g['ve2_W'], g['ve2_b']))

    ei0 = edge_index[0].astype(jnp.int32)
    ei1 = edge_index[1].astype(jnp.int32)
    ea = edge_attr[:, 0].astype(jnp.float32)

    v2c = g['v2c']
    a1 = var_h @ v2c['fl_W'] + v2c['fl_b']
    b1 = cons_h @ v2c['fr_W']
    s1 = _edge_stage(a1, b1, ei1, ei0, ea, v2c['fe_W'],
                     g['ee_W'][0, 0], g['ee_b'][0], _bias_fold(v2c), n_cons)
    cons2 = _bgc_node_post(s1, cons_h, v2c)

    c2v = g['c2v']
    a2 = cons2 @ c2v['fl_W'] + c2v['fl_b']
    b2 = var_h @ c2v['fr_W']
    s2 = _edge_stage(a2, b2, ei0, ei1, ea, c2v['fe_W'],
                     g['ee_W'][0, 0], g['ee_b'][0], _bias_fold(c2v), n_var)
    var2 = _bgc_node_post(s2, var_h, c2v)

    var_g = _relu(_linear(var2, g['ov_W'], g['ov_b']))
    cons_g = _relu(_linear(cons2, g['oc_W'], g['oc_b']))

    x = jnp.concatenate([0.8 * var_g, 0.2 * x1], axis=-1)
    out = _linear(x, p['fc_W'], p['fc_b'])
    return out, var_g, cons_g
